# baseline (speedup 1.0000x reference)
import functools

import jax
import jax.numpy as jnp
import numpy as np
from jax import lax
from jax.experimental import pallas as pl
from jax.experimental.pallas import tpu as pltpu
from jax.experimental.pallas import tpu_sc as plsc

H, W = 240, 304
K = 10
R = 3
TAU = 1000000.0
DELTA_T = 100000.0
GH = (H + K - 1) // K
GW = (W + K - 1) // K
NC = GH * GW
S = 2 * R + 1
B = 8
TPAD = 2048

NSTREAMS = 8
CPS = NC // NSTREAMS
BIN = 64
CNT_LANE = 49
GPAD = 40
GWORDS = CPS * 2 * K * K + 2 * GPAD + 8
HWORDS = CPS * 2 * BIN
HALL = HWORDS + BIN
RECW = 8
WCAP = TPAD + 4

_lut = np.zeros((K * K, BIN), np.float32)
for _ly in range(K):
    for _lx in range(K):
        for _s in range(S * S):
            _r, _c = _ly + _s // S - R, _lx + _s % S - R
            if 0 <= _r < K and 0 <= _c < K:
                _lut[_ly * K + _lx, _s] = 1.0
_LUT = _lut.reshape(-1)


SLAB = 2 * CPS * 2 * S * S
SLABP = 18240


def _body(ev_hbm, len_hbm, lut_hbm, out_hbm, ev_v, len_v, lut_v,
          g_a, g_b, h_a, h_b, wrec, hc_v):
    ci = lax.axis_index("c")
    si = lax.axis_index("s")
    wid = si * 2 + ci
    b = wid // 4
    grp = wid % 4
    lo_a = (grp * 2) * CPS
    lo_b = lo_a + CPS

    pltpu.sync_copy(ev_hbm.at[b], ev_v)
    pltpu.sync_copy(len_hbm, len_v)
    pltpu.sync_copy(lut_hbm, lut_v)

    iota16 = lax.iota(jnp.int32, 16)
    zeros16 = (iota16 * 0).astype(jnp.float32)
    ones16 = zeros16 + 1.0
    lane0 = iota16 == 0
    lenvec = len_v[pl.ds(0, 16)]
    length = jnp.sum(jnp.where(iota16 == b, lenvec, 0))

    def zero8(ref, i, base):
        for u in range(8):
            ref[pl.ds(base + i * 128 + u * 16, 16)] = zeros16

    def zero_g(i, c):
        zero8(g_a, i, 0)
        zero8(g_b, i, 0)
        return c

    lax.fori_loop(0, GWORDS // 128, zero_g, 0)

    def zero_h(i, c):
        zero8(h_a, i, 0)
        zero8(h_b, i, 0)
        return c

    lax.fori_loop(0, HALL // 128, zero_h, 0)

    off_t = []
    for k in range(4):
        s = iota16 + 16 * k
        in49 = s < S * S
        off_t.append(jnp.where(in49, (lax.div(s, S) - R) * K + (lax.rem(s, S) - R), 0))
    cnt_t = ((iota16 + 48) == CNT_LANE).astype(jnp.float32)

    def scan(k, offs):
        off_sa, off_sb = offs
        xi = ev_v[0, pl.ds(k * 16, 16)].astype(jnp.int32)
        yi = ev_v[1, pl.ds(k * 16, 16)].astype(jnp.int32)
        tv = ev_v[2, pl.ds(k * 16, 16)]
        pi = ev_v[3, pl.ds(k * 16, 16)].astype(jnp.int32)
        ch = lax.div(yi, K)
        cw = lax.div(xi, K)
        cid = ch * GW + cw
        lyv = yi - ch * K
        lxv = xi - cw * K
        idxv = k * 16 + iota16
        valid = idxv < length
        e_v = jnp.exp(tv * (1.0 / TAU))
        inv_v = jnp.exp(tv * (-1.0 / TAU))
        cp = cid * 2 + pi
        gq = cp * (K * K) + lyv * K + lxv + GPAD
        hq = cp * BIN
        mbase = (lyv * K + lxv) * BIN

        m_a = valid & (cid >= lo_a) & (cid < lo_a + CPS)
        m_b = valid & (cid >= lo_b) & (cid < lo_b + CPS)
        cs_a = jnp.cumsum(m_a.astype(jnp.int32))
        cs_b = jnp.cumsum(m_b.astype(jnp.int32))
        pos_a = (off_sa + cs_a - 1) * RECW
        pos_b = (WCAP * RECW) + (off_sb + cs_b - 1) * RECW
        m = m_a | m_b
        pos = jnp.where(m_a, pos_a, pos_b)
        lo = jnp.where(m_a, lo_a, lo_b)
        plsc.store_scatter(wrec, [pos], tv, mask=m)
        plsc.store_scatter(wrec, [pos + 1], e_v, mask=m)
        plsc.store_scatter(wrec, [pos + 2], inv_v, mask=m)
        plsc.store_scatter(wrec, [pos + 3],
                           plsc.bitcast(gq - lo * (2 * K * K), jnp.float32),
                           mask=m)
        plsc.store_scatter(wrec, [pos + 4],
                           plsc.bitcast(hq - lo * (2 * BIN), jnp.float32),
                           mask=m)
        plsc.store_scatter(wrec, [pos + 5],
                           plsc.bitcast(mbase, jnp.float32), mask=m)
        return (off_sa + jnp.full((16,), cs_a[15], jnp.int32),
                off_sb + jnp.full((16,), cs_b[15], jnp.int32))

    zi = iota16 * 0
    nwv_a, nwv_b = lax.fori_loop(0, TPAD // 16, scan, (zi, zi))
    nw_a = nwv_a[0]
    nw_b = nwv_b[0]

    r8 = lax.rem(iota16, 8)
    fpart = jnp.where(r8 == 0, -1e30, jnp.where(r8 == 1, 1.0, 0.0))
    ipart = jnp.where(r8 == 3, GPAD, jnp.where(r8 == 4, HWORDS, 0))
    dummy = jnp.where(r8 < 3, fpart, plsc.bitcast(ipart, jnp.float32))
    wrec[pl.ds(nw_a * RECW, 16)] = dummy
    wrec[pl.ds((WCAP + nw_b) * RECW, 16)] = dummy

    ctr_t = ((iota16 + 16) == (R * S + R)).astype(jnp.float32)

    def step(rbase, g_v, h_v, i, nw, carry):
        L, texp = carry
        ii = rbase + jnp.minimum(i, nw)
        v = wrec[pl.ds(ii * RECW, 16)]
        vi = plsc.bitcast(v, jnp.int32)
        cutoff = v[0] - DELTA_T

        def cond(c):
            return c[1] < cutoff

        def expire(c):
            Lc, _ = c
            rv = wrec[pl.ds((rbase + Lc) * RECW, 16)]
            rvi = plsc.bitcast(rv, jnp.int32)
            plsc.addupdate_scatter(
                g_v, [jnp.full((16,), rvi[3], jnp.int32)],
                zeros16 - jnp.full((16,), rv[1], jnp.float32), mask=lane0)
            nxt = wrec[pl.ds((rbase + Lc + 1) * RECW, 16)]
            return (Lc + 1, nxt[0])

        L, texp = lax.while_loop(cond, expire, (L, texp))

        giv = jnp.full((16,), vi[3], jnp.int32)
        invv = jnp.full((16,), v[2], jnp.float32)
        hbv = jnp.full((16,), vi[4], jnp.int32) + iota16
        mbv = jnp.full((16,), vi[5], jnp.int32) + iota16
        for k in range(4):
            gval = plsc.load_gather(g_v, [giv + off_t[k]])
            mk = plsc.load_gather(lut_v, [mbv + 16 * k])
            vals = gval * (mk * invv)
            if k == 1:
                vals = vals + ctr_t
            if k == 3:
                vals = vals + cnt_t
            plsc.addupdate_scatter(h_v, [hbv + 16 * k], vals)
        plsc.addupdate_scatter(
            g_v, [giv], jnp.full((16,), v[1], jnp.float32), mask=lane0)
        return (L, texp)

    texp_a = wrec[pl.ds(0, 16)][0]
    texp_b = wrec[pl.ds(WCAP * RECW, 16)][0]

    def proc(i, carry):
        ca, cb = carry
        ca = step(0, g_a, h_a, i, nw_a, ca)
        cb = step(WCAP, g_b, h_b, i, nw_b, cb)
        return (ca, cb)

    lax.fori_loop(0, jnp.maximum(nw_a, nw_b), proc,
                  ((jnp.int32(0), texp_a), (jnp.int32(0), texp_b)))

    def norm1(h_v, sbase):
        def norm(c, carry):
            cnt = (h_v[pl.ds(c * (2 * BIN) + 48, 16)][CNT_LANE - 48]
                   + h_v[pl.ds(c * (2 * BIN) + BIN + 48, 16)][CNT_LANE - 48])
            scale = ones16 / jnp.full((16,), cnt + 1e-6, jnp.float32)
            for pp in range(2):
                for k in range(4):
                    v = h_v[pl.ds(c * (2 * BIN) + pp * BIN + k * 16, 16)]
                    hc_v[pl.ds(sbase + c * (2 * S * S) + pp * (S * S)
                               + k * 16, 16)] = v * scale
            return carry

        lax.fori_loop(0, CPS, norm, 0)

    norm1(h_a, 0)
    norm1(h_b, CPS * 2 * S * S)

    pltpu.sync_copy(hc_v.at[pl.ds(0, SLABP)],
                    out_hbm.at[pl.ds(wid * SLABP, SLABP)])


@jax.jit
def _hats_sc(comp, len16, lut):
    mesh = plsc.VectorSubcoreMesh(core_axis_name="c", subcore_axis_name="s",
                                  num_cores=2, num_subcores=16)
    f = pl.kernel(
        _body,
        out_type=jax.ShapeDtypeStruct((4 * B * SLABP,), jnp.float32),
        mesh=mesh,
        compiler_params=pltpu.CompilerParams(needs_layout_passes=False),
        scratch_types=[
            pltpu.VMEM((4, TPAD), jnp.float32),
            pltpu.VMEM((16,), jnp.int32),
            pltpu.VMEM((K * K * BIN,), jnp.float32),
            pltpu.VMEM((GWORDS,), jnp.float32),
            pltpu.VMEM((GWORDS,), jnp.float32),
            pltpu.VMEM((HALL,), jnp.float32),
            pltpu.VMEM((HALL,), jnp.float32),
            pltpu.VMEM((2 * WCAP * RECW,), jnp.float32),
            pltpu.VMEM((SLABP + 16,), jnp.float32),
        ],
    )
    return f(comp, len16, lut)


def kernel(events, lengths):
    comp = jnp.transpose(events, (0, 2, 1))
    len16 = jnp.zeros((16,), jnp.int32).at[:B].set(lengths.astype(jnp.int32))
    flat = _hats_sc(comp, len16, jnp.asarray(_LUT))
    out = flat.reshape(4 * B, SLABP)[:, :SLAB]
    return out.reshape(B, NC, 2, S, S)


# speedup vs baseline: 1.1202x; 1.1202x over previous
import functools

import jax
import jax.numpy as jnp
import numpy as np
from jax import lax
from jax.experimental import pallas as pl
from jax.experimental.pallas import tpu as pltpu
from jax.experimental.pallas import tpu_sc as plsc

H, W = 240, 304
K = 10
R = 3
TAU = 1000000.0
DELTA_T = 100000.0
GH = (H + K - 1) // K
GW = (W + K - 1) // K
NC = GH * GW
S = 2 * R + 1
B = 8
TPAD = 2048

CPSM = 94
BIN = 64
CNT_LANE = 49
GPAD = 40
GWORDS = 18944
HWORDS = CPSM * 2 * BIN
HALL = HWORDS + 2 * BIN
RECW = 8
WCAP = TPAD + 4
HCW = 2 * CPSM * 2 * S * S + 16

_lut = np.zeros((K * K, BIN), np.float32)
for _ly in range(K):
    for _lx in range(K):
        for _s in range(S * S):
            _r, _c = _ly + _s // S - R, _lx + _s % S - R
            if 0 <= _r < K and 0 <= _c < K:
                _lut[_ly * K + _lx, _s] = 1.0
_LUT = _lut.reshape(-1)


def _body(ev_hbm, len_hbm, lut_hbm, out_hbm, ev_v, len_v, lut_v,
          g_a, g_b, h_a, h_b, wrec, hc_v):
    ci = lax.axis_index("c")
    si = lax.axis_index("s")
    wid = si * 2 + ci
    b = wid // 4
    grp = wid % 4
    cps = 94 - (grp % 2) * 2
    lo_a = 188 * ((grp + 1) // 2) + 184 * (grp // 2)
    lo_b = lo_a + cps

    pltpu.sync_copy(ev_hbm.at[b], ev_v)
    pltpu.sync_copy(len_hbm, len_v)
    pltpu.sync_copy(lut_hbm, lut_v)

    iota16 = lax.iota(jnp.int32, 16)
    zeros16 = (iota16 * 0).astype(jnp.float32)
    ones16 = zeros16 + 1.0
    lane0 = iota16 == 0
    lenvec = len_v[pl.ds(0, 16)]
    length = jnp.sum(jnp.where(iota16 == b, lenvec, 0))

    def zero8(ref, i, base):
        for u in range(8):
            ref[pl.ds(base + i * 128 + u * 16, 16)] = zeros16

    def zero_g(i, c):
        zero8(g_a, i, 0)
        zero8(g_b, i, 0)
        return c

    lax.fori_loop(0, GWORDS // 128, zero_g, 0)

    def zero_h(i, c):
        zero8(h_a, i, 0)
        zero8(h_b, i, 0)
        return c

    lax.fori_loop(0, HALL // 128, zero_h, 0)

    off_t = []
    for k in range(4):
        s = iota16 + 16 * k
        in49 = s < S * S
        off_t.append(jnp.where(in49, (lax.div(s, S) - R) * K + (lax.rem(s, S) - R), 0))
    cnt_t = ((iota16 + 48) == CNT_LANE).astype(jnp.float32)

    def scan(k, offs):
        off_sa, off_sb = offs
        xi = ev_v[0, pl.ds(k * 16, 16)].astype(jnp.int32)
        yi = ev_v[1, pl.ds(k * 16, 16)].astype(jnp.int32)
        tv = ev_v[2, pl.ds(k * 16, 16)]
        pi = ev_v[3, pl.ds(k * 16, 16)].astype(jnp.int32)
        ch = lax.div(yi, K)
        cw = lax.div(xi, K)
        cid = ch * GW + cw
        lyv = yi - ch * K
        lxv = xi - cw * K
        idxv = k * 16 + iota16
        valid = idxv < length
        e_v = jnp.exp(tv * (1.0 / TAU))
        inv_v = jnp.exp(tv * (-1.0 / TAU))
        cp = cid * 2 + pi
        gq = cp * (K * K) + lyv * K + lxv + GPAD
        hq = cp * BIN
        mbase = (lyv * K + lxv) * BIN

        m_a = valid & (cid >= lo_a) & (cid < lo_a + cps)
        m_b = valid & (cid >= lo_b) & (cid < lo_b + cps)
        cs_a = jnp.cumsum(m_a.astype(jnp.int32))
        cs_b = jnp.cumsum(m_b.astype(jnp.int32))
        pos_a = (off_sa + cs_a - 1) * RECW
        pos_b = (WCAP * RECW) + (off_sb + cs_b - 1) * RECW
        m = m_a | m_b
        pos = jnp.where(m_a, pos_a, pos_b)
        lo = jnp.where(m_a, lo_a, lo_b)
        plsc.store_scatter(wrec, [pos], tv, mask=m)
        plsc.store_scatter(wrec, [pos + 1], e_v, mask=m)
        plsc.store_scatter(wrec, [pos + 2], inv_v, mask=m)
        plsc.store_scatter(wrec, [pos + 3],
                           plsc.bitcast(gq - lo * (2 * K * K), jnp.float32),
                           mask=m)
        plsc.store_scatter(wrec, [pos + 4],
                           plsc.bitcast(hq - lo * (2 * BIN), jnp.float32),
                           mask=m)
        plsc.store_scatter(wrec, [pos + 5],
                           plsc.bitcast(mbase, jnp.float32), mask=m)
        return (off_sa + jnp.full((16,), cs_a[15], jnp.int32),
                off_sb + jnp.full((16,), cs_b[15], jnp.int32))

    zi = iota16 * 0
    nwv_a, nwv_b = lax.fori_loop(0, TPAD // 16, scan, (zi, zi))
    nw_a = nwv_a[0]
    nw_b = nwv_b[0]

    r8 = lax.rem(iota16, 8)
    fpart = jnp.where(r8 == 0, -1e30, jnp.where(r8 == 1, 1.0, 0.0))
    ipart = jnp.where(r8 == 3, GPAD, jnp.where(r8 == 4, HWORDS, 0))
    dummy = jnp.where(r8 < 3, fpart, plsc.bitcast(ipart, jnp.float32))
    wrec[pl.ds(nw_a * RECW, 16)] = dummy
    wrec[pl.ds((WCAP + nw_b) * RECW, 16)] = dummy

    ctr_t = ((iota16 + 16) == (R * S + R)).astype(jnp.float32)

    def step(rbase, g_v, h_v, i, nw, carry):
        L, texp = carry
        ii = rbase + jnp.minimum(i, nw)
        v = wrec[pl.ds(ii * RECW, 16)]
        vi = plsc.bitcast(v, jnp.int32)
        cutoff = v[0] - DELTA_T

        def cond(c):
            return c[1] < cutoff

        def expire(c):
            Lc, _ = c
            rv = wrec[pl.ds((rbase + Lc) * RECW, 16)]
            rvi = plsc.bitcast(rv, jnp.int32)
            plsc.addupdate_scatter(
                g_v, [jnp.full((16,), rvi[3], jnp.int32)],
                zeros16 - jnp.full((16,), rv[1], jnp.float32), mask=lane0)
            nxt = wrec[pl.ds((rbase + Lc + 1) * RECW, 16)]
            return (Lc + 1, nxt[0])

        L, texp = lax.while_loop(cond, expire, (L, texp))

        giv = jnp.full((16,), vi[3], jnp.int32)
        invv = jnp.full((16,), v[2], jnp.float32)
        hbv = jnp.full((16,), vi[4], jnp.int32) + iota16
        mbv = jnp.full((16,), vi[5], jnp.int32) + iota16
        for k in range(4):
            gval = plsc.load_gather(g_v, [giv + off_t[k]])
            mk = plsc.load_gather(lut_v, [mbv + 16 * k])
            vals = gval * (mk * invv)
            if k == 1:
                vals = vals + ctr_t
            if k == 3:
                vals = vals + cnt_t
            plsc.addupdate_scatter(h_v, [hbv + 16 * k], vals)
        plsc.addupdate_scatter(
            g_v, [giv], jnp.full((16,), v[1], jnp.float32), mask=lane0)
        return (L, texp)

    texp_a = wrec[pl.ds(0, 16)][0]
    texp_b = wrec[pl.ds(WCAP * RECW, 16)][0]

    def proc(i, carry):
        ca, cb = carry
        ca = step(0, g_a, h_a, i, nw_a, ca)
        cb = step(WCAP, g_b, h_b, i, nw_b, cb)
        return (ca, cb)

    lax.fori_loop(0, jnp.maximum(nw_a, nw_b), proc,
                  ((jnp.int32(0), texp_a), (jnp.int32(0), texp_b)))

    def norm1(h_v, sbase):
        def norm(c, carry):
            cnt = (h_v[pl.ds(c * (2 * BIN) + 48, 16)][CNT_LANE - 48]
                   + h_v[pl.ds(c * (2 * BIN) + BIN + 48, 16)][CNT_LANE - 48])
            scale = ones16 / jnp.full((16,), cnt + 1e-6, jnp.float32)
            for pp in range(2):
                for k in range(4):
                    v = h_v[pl.ds(c * (2 * BIN) + pp * BIN + k * 16, 16)]
                    hc_v[pl.ds(sbase + c * (2 * S * S) + pp * (S * S)
                               + k * 16, 16)] = v * scale
            return carry

        lax.fori_loop(0, cps, norm, 0)

    norm1(h_a, 0)
    norm1(h_b, cps * (2 * S * S))

    base = b * (NC * 2 * S * S) + lo_a * (2 * S * S)

    @pl.when(cps == 94)
    def _():
        pltpu.sync_copy(hc_v.at[pl.ds(0, 188 * 98)],
                        out_hbm.at[pl.ds(base, 188 * 98)])

    @pl.when(cps == 92)
    def _():
        pltpu.sync_copy(hc_v.at[pl.ds(0, 184 * 98)],
                        out_hbm.at[pl.ds(base, 184 * 98)])


@jax.jit
def _hats_sc(comp, len16, lut):
    mesh = plsc.VectorSubcoreMesh(core_axis_name="c", subcore_axis_name="s",
                                  num_cores=2, num_subcores=16)
    f = pl.kernel(
        _body,
        out_type=jax.ShapeDtypeStruct((B * NC * 2 * S * S,), jnp.float32),
        mesh=mesh,
        compiler_params=pltpu.CompilerParams(needs_layout_passes=False),
        scratch_types=[
            pltpu.VMEM((4, TPAD), jnp.float32),
            pltpu.VMEM((16,), jnp.int32),
            pltpu.VMEM((K * K * BIN,), jnp.float32),
            pltpu.VMEM((GWORDS,), jnp.float32),
            pltpu.VMEM((GWORDS,), jnp.float32),
            pltpu.VMEM((HALL,), jnp.float32),
            pltpu.VMEM((HALL,), jnp.float32),
            pltpu.VMEM((2 * WCAP * RECW,), jnp.float32),
            pltpu.VMEM((HCW,), jnp.float32),
        ],
    )
    return f(comp, len16, lut)


def kernel(events, lengths):
    comp = jnp.transpose(events, (0, 2, 1))
    len16 = jnp.zeros((16,), jnp.int32).at[:B].set(lengths.astype(jnp.int32))
    flat = _hats_sc(comp, len16, jnp.asarray(_LUT))
    return flat.reshape(B, NC, 2, S, S)


# speedup vs baseline: 1.6060x; 1.4336x over previous
import functools

import jax
import jax.numpy as jnp
import numpy as np
from jax import lax
from jax.experimental import pallas as pl
from jax.experimental.pallas import tpu as pltpu
from jax.experimental.pallas import tpu_sc as plsc

H, W = 240, 304
K = 10
R = 3
TAU = 1000000.0
DELTA_T = 100000.0
GH = (H + K - 1) // K
GW = (W + K - 1) // K
NC = GH * GW
S = 2 * R + 1
B = 8
TPAD = 2048

NSTREAMS = 8
CPS = NC // NSTREAMS
BIN = 64
CNT_LANE = 49
GPAD = 40
GWORDS = CPS * 2 * K * K + 2 * GPAD + 8
HWORDS = CPS * 2 * BIN
HALL = HWORDS + BIN
RECW = 8
WCAP = TPAD + 4

_lut = np.zeros((K * K, BIN), np.float32)
for _ly in range(K):
    for _lx in range(K):
        for _s in range(S * S):
            _r, _c = _ly + _s // S - R, _lx + _s % S - R
            if 0 <= _r < K and 0 <= _c < K:
                _lut[_ly * K + _lx, _s] = 1.0
_LUT = _lut.reshape(-1)


def _body(ev_hbm, len_hbm, lut_hbm, out_hbm, ev_v, len_v, lut_v,
          g_a, g_b, h_a, h_b, wrec):
    ci = lax.axis_index("c")
    si = lax.axis_index("s")
    wid = si * 2 + ci
    b = wid // 4
    grp = wid % 4
    lo_a = (grp * 2) * CPS
    lo_b = lo_a + CPS

    pltpu.sync_copy(ev_hbm.at[b], ev_v)
    pltpu.sync_copy(len_hbm, len_v)
    pltpu.sync_copy(lut_hbm, lut_v)

    iota16 = lax.iota(jnp.int32, 16)
    zeros16 = (iota16 * 0).astype(jnp.float32)
    ones16 = zeros16 + 1.0
    lane0 = iota16 == 0
    lenvec = len_v[pl.ds(0, 16)]
    length = jnp.sum(jnp.where(iota16 == b, lenvec, 0))

    def zero8(ref, i, base):
        for u in range(8):
            ref[pl.ds(base + i * 128 + u * 16, 16)] = zeros16

    def zero_g(i, c):
        zero8(g_a, i, 0)
        zero8(g_b, i, 0)
        return c

    lax.fori_loop(0, GWORDS // 128, zero_g, 0)

    def zero_h(i, c):
        zero8(h_a, i, 0)
        zero8(h_b, i, 0)
        return c

    lax.fori_loop(0, HALL // 128, zero_h, 0)

    off_t = []
    for k in range(4):
        s = iota16 + 16 * k
        in49 = s < S * S
        off_t.append(jnp.where(in49, (lax.div(s, S) - R) * K + (lax.rem(s, S) - R), 0))
    cnt_t = ((iota16 + 48) == CNT_LANE).astype(jnp.float32)

    def scan(k, offs):
        off_sa, off_sb = offs
        xi = ev_v[0, pl.ds(k * 16, 16)].astype(jnp.int32)
        yi = ev_v[1, pl.ds(k * 16, 16)].astype(jnp.int32)
        tv = ev_v[2, pl.ds(k * 16, 16)]
        pi = ev_v[3, pl.ds(k * 16, 16)].astype(jnp.int32)
        ch = lax.div(yi, K)
        cw = lax.div(xi, K)
        cid = ch * GW + cw
        lyv = yi - ch * K
        lxv = xi - cw * K
        idxv = k * 16 + iota16
        valid = idxv < length
        e_v = jnp.exp(tv * (1.0 / TAU))
        inv_v = jnp.exp(tv * (-1.0 / TAU))
        cp = cid * 2 + pi
        gq = cp * (K * K) + lyv * K + lxv + GPAD
        hq = cp * BIN
        mbase = (lyv * K + lxv) * BIN

        m_a = valid & (cid >= lo_a) & (cid < lo_a + CPS)
        m_b = valid & (cid >= lo_b) & (cid < lo_b + CPS)
        cs_a = jnp.cumsum(m_a.astype(jnp.int32))
        cs_b = jnp.cumsum(m_b.astype(jnp.int32))
        pos_a = (off_sa + cs_a - 1) * RECW
        pos_b = (WCAP * RECW) + (off_sb + cs_b - 1) * RECW
        m = m_a | m_b
        pos = jnp.where(m_a, pos_a, pos_b)
        lo = jnp.where(m_a, lo_a, lo_b)
        plsc.store_scatter(wrec, [pos], tv, mask=m)
        plsc.store_scatter(wrec, [pos + 1], e_v, mask=m)
        plsc.store_scatter(wrec, [pos + 2], inv_v, mask=m)
        plsc.store_scatter(wrec, [pos + 3],
                           plsc.bitcast(gq - lo * (2 * K * K), jnp.float32),
                           mask=m)
        plsc.store_scatter(wrec, [pos + 4],
                           plsc.bitcast(hq - lo * (2 * BIN), jnp.float32),
                           mask=m)
        plsc.store_scatter(wrec, [pos + 5],
                           plsc.bitcast(mbase, jnp.float32), mask=m)
        return (off_sa + jnp.full((16,), cs_a[15], jnp.int32),
                off_sb + jnp.full((16,), cs_b[15], jnp.int32))

    zi = iota16 * 0
    nwv_a, nwv_b = lax.fori_loop(0, TPAD // 16, scan, (zi, zi))
    nw_a = nwv_a[0]
    nw_b = nwv_b[0]

    r8 = lax.rem(iota16, 8)
    fpart = jnp.where(r8 == 0, -1e30, jnp.where(r8 == 1, 1.0, 0.0))
    ipart = jnp.where(r8 == 3, GPAD, jnp.where(r8 == 4, HWORDS, 0))
    dummy = jnp.where(r8 < 3, fpart, plsc.bitcast(ipart, jnp.float32))
    wrec[pl.ds(nw_a * RECW, 16)] = dummy
    wrec[pl.ds((WCAP + nw_b) * RECW, 16)] = dummy

    ctr_t = ((iota16 + 16) == (R * S + R)).astype(jnp.float32)

    def step(rbase, g_v, h_v, i, nw, carry):
        L, texp = carry
        ii = rbase + jnp.minimum(i, nw)
        v = wrec[pl.ds(ii * RECW, 16)]
        vi = plsc.bitcast(v, jnp.int32)
        cutoff = v[0] - DELTA_T

        def cond(c):
            return c[1] < cutoff

        def expire(c):
            Lc, _ = c
            rv = wrec[pl.ds((rbase + Lc) * RECW, 16)]
            rvi = plsc.bitcast(rv, jnp.int32)
            plsc.addupdate_scatter(
                g_v, [jnp.full((16,), rvi[3], jnp.int32)],
                zeros16 - jnp.full((16,), rv[1], jnp.float32), mask=lane0)
            nxt = wrec[pl.ds((rbase + Lc + 1) * RECW, 16)]
            return (Lc + 1, nxt[0])

        L, texp = lax.while_loop(cond, expire, (L, texp))

        giv = jnp.full((16,), vi[3], jnp.int32)
        invv = jnp.full((16,), v[2], jnp.float32)
        hbv = jnp.full((16,), vi[4], jnp.int32) + iota16
        mbv = jnp.full((16,), vi[5], jnp.int32) + iota16
        for k in range(4):
            gval = plsc.load_gather(g_v, [giv + off_t[k]])
            mk = plsc.load_gather(lut_v, [mbv + 16 * k])
            vals = gval * (mk * invv)
            if k == 1:
                vals = vals + ctr_t
            if k == 3:
                vals = vals + cnt_t
            plsc.addupdate_scatter(h_v, [hbv + 16 * k], vals)
        plsc.addupdate_scatter(
            g_v, [giv], jnp.full((16,), v[1], jnp.float32), mask=lane0)
        return (L, texp)

    texp_a = wrec[pl.ds(0, 16)][0]
    texp_b = wrec[pl.ds(WCAP * RECW, 16)][0]

    def proc(i, carry):
        ca, cb = carry
        ca = step(0, g_a, h_a, i, nw_a, ca)
        cb = step(WCAP, g_b, h_b, i, nw_b, cb)
        return (ca, cb)

    lax.fori_loop(0, jnp.maximum(nw_a, nw_b), proc,
                  ((jnp.int32(0), texp_a), (jnp.int32(0), texp_b)))

    def norm1(h_v, c):
        cnt = (h_v[pl.ds(c * (2 * BIN) + 48, 16)][CNT_LANE - 48]
               + h_v[pl.ds(c * (2 * BIN) + BIN + 48, 16)][CNT_LANE - 48])
        scale = ones16 / jnp.full((16,), cnt + 1e-6, jnp.float32)
        for k in range(2 * BIN // 16):
            sl = pl.ds(c * (2 * BIN) + k * 16, 16)
            h_v[sl] = h_v[sl] * scale

    def norm(c, carry):
        norm1(h_a, c)
        norm1(h_b, c)
        return carry

    lax.fori_loop(0, CPS, norm, 0)

    base = (b * NC + lo_a) * (2 * BIN)
    pltpu.sync_copy(h_a.at[pl.ds(0, HWORDS)], out_hbm.at[pl.ds(base, HWORDS)])
    pltpu.sync_copy(h_b.at[pl.ds(0, HWORDS)],
                    out_hbm.at[pl.ds(base + HWORDS, HWORDS)])


@jax.jit
def _hats_sc(comp, len16, lut):
    mesh = plsc.VectorSubcoreMesh(core_axis_name="c", subcore_axis_name="s",
                                  num_cores=2, num_subcores=16)
    f = pl.kernel(
        _body,
        out_type=jax.ShapeDtypeStruct((B * NC * 2 * BIN,), jnp.float32),
        mesh=mesh,
        compiler_params=pltpu.CompilerParams(needs_layout_passes=False),
        scratch_types=[
            pltpu.VMEM((4, TPAD), jnp.float32),
            pltpu.VMEM((16,), jnp.int32),
            pltpu.VMEM((K * K * BIN,), jnp.float32),
            pltpu.VMEM((GWORDS,), jnp.float32),
            pltpu.VMEM((GWORDS,), jnp.float32),
            pltpu.VMEM((HALL,), jnp.float32),
            pltpu.VMEM((HALL,), jnp.float32),
            pltpu.VMEM((2 * WCAP * RECW,), jnp.float32),
        ],
    )
    return f(comp, len16, lut)


def kernel(events, lengths):
    comp = jnp.transpose(events, (0, 2, 1))
    len16 = jnp.zeros((16,), jnp.int32).at[:B].set(lengths.astype(jnp.int32))
    flat = _hats_sc(comp, len16, jnp.asarray(_LUT))
    out = flat.reshape(B, NC, 2, BIN)[..., :S * S]
    return out.reshape(B, NC, 2, S, S)


# speedup vs baseline: 1.6847x; 1.0490x over previous
import functools

import jax
import jax.numpy as jnp
import numpy as np
from jax import lax
from jax.experimental import pallas as pl
from jax.experimental.pallas import tpu as pltpu
from jax.experimental.pallas import tpu_sc as plsc

H, W = 240, 304
K = 10
R = 3
TAU = 1000000.0
DELTA_T = 100000.0
GH = (H + K - 1) // K
GW = (W + K - 1) // K
NC = GH * GW
S = 2 * R + 1
B = 8
TPAD = 2048

NSTREAMS = 8
CPS = NC // NSTREAMS
BIN = 64
CNT_LANE = 49
GPAD = 40
GWORDS = CPS * 2 * K * K + 2 * GPAD + 8
HWORDS = CPS * 2 * BIN
HALL = HWORDS + BIN
RECW = 8
WCAP = TPAD + 4

_lut = np.zeros((K * K, BIN), np.float32)
for _ly in range(K):
    for _lx in range(K):
        for _s in range(S * S):
            _r, _c = _ly + _s // S - R, _lx + _s % S - R
            if 0 <= _r < K and 0 <= _c < K:
                _lut[_ly * K + _lx, _s] = 1.0
_LUT = _lut.reshape(-1)


def _body(ev_hbm, len_hbm, lut_hbm, out_hbm, ev_v, len_v, lut_v,
          g_a, g_b, h_a, h_b, wrec):
    ci = lax.axis_index("c")
    si = lax.axis_index("s")
    wid = si * 2 + ci
    b = wid // 4
    grp = wid % 4
    lo_a = (grp * 2) * CPS
    lo_b = lo_a + CPS

    pltpu.sync_copy(ev_hbm.at[b], ev_v)
    pltpu.sync_copy(len_hbm, len_v)
    pltpu.sync_copy(lut_hbm, lut_v)

    iota16 = lax.iota(jnp.int32, 16)
    zeros16 = (iota16 * 0).astype(jnp.float32)
    ones16 = zeros16 + 1.0
    lane0 = iota16 == 0
    lenvec = len_v[pl.ds(0, 16)]
    length = jnp.sum(jnp.where(iota16 == b, lenvec, 0))

    def zero8(ref, i, base):
        for u in range(8):
            ref[pl.ds(base + i * 128 + u * 16, 16)] = zeros16

    def zero_g(i, c):
        zero8(g_a, i, 0)
        zero8(g_b, i, 0)
        return c

    lax.fori_loop(0, GWORDS // 128, zero_g, 0)

    def zero_h(i, c):
        zero8(h_a, i, 0)
        zero8(h_b, i, 0)
        return c

    lax.fori_loop(0, HALL // 128, zero_h, 0)

    off_t = []
    for k in range(4):
        s = iota16 + 16 * k
        in49 = s < S * S
        off_t.append(jnp.where(in49, (lax.div(s, S) - R) * K + (lax.rem(s, S) - R), 0))
    cnt_t = ((iota16 + 48) == CNT_LANE).astype(jnp.float32)

    def scan(k, offs):
        off_sa, off_sb = offs
        xi = ev_v[0, pl.ds(k * 16, 16)].astype(jnp.int32)
        yi = ev_v[1, pl.ds(k * 16, 16)].astype(jnp.int32)
        tv = ev_v[2, pl.ds(k * 16, 16)]
        pi = ev_v[3, pl.ds(k * 16, 16)].astype(jnp.int32)
        ch = lax.div(yi, K)
        cw = lax.div(xi, K)
        cid = ch * GW + cw
        lyv = yi - ch * K
        lxv = xi - cw * K
        idxv = k * 16 + iota16
        valid = idxv < length
        e_v = jnp.exp(tv * (1.0 / TAU))
        inv_v = jnp.exp(tv * (-1.0 / TAU))
        cp = cid * 2 + pi
        gq = cp * (K * K) + lyv * K + lxv + GPAD
        hq = cp * BIN
        mbase = (lyv * K + lxv) * BIN

        m_a = valid & (cid >= lo_a) & (cid < lo_a + CPS)
        m_b = valid & (cid >= lo_b) & (cid < lo_b + CPS)
        cs_a = jnp.cumsum(m_a.astype(jnp.int32))
        cs_b = jnp.cumsum(m_b.astype(jnp.int32))
        pos_a = (off_sa + cs_a - 1) * RECW
        pos_b = (WCAP * RECW) + (off_sb + cs_b - 1) * RECW
        m = m_a | m_b
        pos = jnp.where(m_a, pos_a, pos_b)
        lo = jnp.where(m_a, lo_a, lo_b)
        plsc.store_scatter(wrec, [pos], tv, mask=m)
        plsc.store_scatter(wrec, [pos + 1], e_v, mask=m)
        plsc.store_scatter(wrec, [pos + 2], inv_v, mask=m)
        plsc.store_scatter(wrec, [pos + 3],
                           plsc.bitcast(gq - lo * (2 * K * K), jnp.float32),
                           mask=m)
        plsc.store_scatter(wrec, [pos + 4],
                           plsc.bitcast(hq - lo * (2 * BIN), jnp.float32),
                           mask=m)
        plsc.store_scatter(wrec, [pos + 5],
                           plsc.bitcast(mbase, jnp.float32), mask=m)
        return (off_sa + jnp.full((16,), cs_a[15], jnp.int32),
                off_sb + jnp.full((16,), cs_b[15], jnp.int32))

    zi = iota16 * 0
    nwv_a, nwv_b = lax.fori_loop(0, TPAD // 16, scan, (zi, zi))
    nw_a = nwv_a[0]
    nw_b = nwv_b[0]

    r8 = lax.rem(iota16, 8)
    fpart = jnp.where(r8 == 0, -1e30, jnp.where(r8 == 1, 1.0, 0.0))
    ipart = jnp.where(r8 == 3, GPAD, jnp.where(r8 == 4, HWORDS, 0))
    dummy = jnp.where(r8 < 3, fpart, plsc.bitcast(ipart, jnp.float32))
    wrec[pl.ds(nw_a * RECW, 16)] = dummy
    wrec[pl.ds((WCAP + nw_b) * RECW, 16)] = dummy

    ctr_t = ((iota16 + 16) == (R * S + R)).astype(jnp.float32)

    def step(rbase, g_v, h_v, i, nw, carry):
        L, texp = carry
        ii = rbase + jnp.minimum(2 * i, nw)
        v16 = wrec[pl.ds(ii * RECW, 16)]
        vi16 = plsc.bitcast(v16, jnp.int32)
        for half in (0, 8):
            cutoff = v16[half] - DELTA_T

            def cond(c):
                return c[1] < cutoff

            def expire(c):
                Lc, _ = c
                rv = wrec[pl.ds((rbase + Lc) * RECW, 16)]
                rvi = plsc.bitcast(rv, jnp.int32)
                plsc.addupdate_scatter(
                    g_v, [jnp.full((16,), rvi[3], jnp.int32)],
                    zeros16 - jnp.full((16,), rv[1], jnp.float32), mask=lane0)
                nxt = wrec[pl.ds((rbase + Lc + 1) * RECW, 16)]
                return (Lc + 1, nxt[0])

            L, texp = lax.while_loop(cond, expire, (L, texp))

            giv = jnp.full((16,), vi16[half + 3], jnp.int32)
            invv = jnp.full((16,), v16[half + 2], jnp.float32)
            hbv = jnp.full((16,), vi16[half + 4], jnp.int32) + iota16
            mbv = jnp.full((16,), vi16[half + 5], jnp.int32) + iota16
            for k in range(4):
                gval = plsc.load_gather(g_v, [giv + off_t[k]])
                mk = plsc.load_gather(lut_v, [mbv + 16 * k])
                vals = gval * (mk * invv)
                if k == 1:
                    vals = vals + ctr_t
                if k == 3:
                    vals = vals + cnt_t
                plsc.addupdate_scatter(h_v, [hbv + 16 * k], vals)
            plsc.addupdate_scatter(
                g_v, [giv], jnp.full((16,), v16[half + 1], jnp.float32),
                mask=lane0)
        return (L, texp)

    texp_a = wrec[pl.ds(0, 16)][0]
    texp_b = wrec[pl.ds(WCAP * RECW, 16)][0]

    def proc(i, carry):
        ca, cb = carry
        ca = step(0, g_a, h_a, i, nw_a, ca)
        cb = step(WCAP, g_b, h_b, i, nw_b, cb)
        return (ca, cb)

    lax.fori_loop(0, lax.div(jnp.maximum(nw_a, nw_b) + 1, 2), proc,
                  ((jnp.int32(0), texp_a), (jnp.int32(0), texp_b)))

    def norm1(h_v, c):
        cnt = (h_v[pl.ds(c * (2 * BIN) + 48, 16)][CNT_LANE - 48]
               + h_v[pl.ds(c * (2 * BIN) + BIN + 48, 16)][CNT_LANE - 48])
        scale = ones16 / jnp.full((16,), cnt + 1e-6, jnp.float32)
        for k in range(2 * BIN // 16):
            sl = pl.ds(c * (2 * BIN) + k * 16, 16)
            h_v[sl] = h_v[sl] * scale

    def norm(c, carry):
        norm1(h_a, c)
        norm1(h_b, c)
        return carry

    lax.fori_loop(0, CPS, norm, 0)

    base = (b * NC + lo_a) * (2 * BIN)
    pltpu.sync_copy(h_a.at[pl.ds(0, HWORDS)], out_hbm.at[pl.ds(base, HWORDS)])
    pltpu.sync_copy(h_b.at[pl.ds(0, HWORDS)],
                    out_hbm.at[pl.ds(base + HWORDS, HWORDS)])


@jax.jit
def _hats_sc(comp, len16, lut):
    mesh = plsc.VectorSubcoreMesh(core_axis_name="c", subcore_axis_name="s",
                                  num_cores=2, num_subcores=16)
    f = pl.kernel(
        _body,
        out_type=jax.ShapeDtypeStruct((B * NC * 2 * BIN,), jnp.float32),
        mesh=mesh,
        compiler_params=pltpu.CompilerParams(needs_layout_passes=False),
        scratch_types=[
            pltpu.VMEM((4, TPAD), jnp.float32),
            pltpu.VMEM((16,), jnp.int32),
            pltpu.VMEM((K * K * BIN,), jnp.float32),
            pltpu.VMEM((GWORDS,), jnp.float32),
            pltpu.VMEM((GWORDS,), jnp.float32),
            pltpu.VMEM((HALL,), jnp.float32),
            pltpu.VMEM((HALL,), jnp.float32),
            pltpu.VMEM((2 * WCAP * RECW,), jnp.float32),
        ],
    )
    return f(comp, len16, lut)


def kernel(events, lengths):
    comp = jnp.transpose(events, (0, 2, 1))
    len16 = jnp.zeros((16,), jnp.int32).at[:B].set(lengths.astype(jnp.int32))
    flat = _hats_sc(comp, len16, jnp.asarray(_LUT))
    out = flat.reshape(B, NC, 2, BIN)[..., :S * S]
    return out.reshape(B, NC, 2, S, S)


# speedup vs baseline: 1.7193x; 1.0206x over previous
import functools

import jax
import jax.numpy as jnp
import numpy as np
from jax import lax
from jax.experimental import pallas as pl
from jax.experimental.pallas import tpu as pltpu
from jax.experimental.pallas import tpu_sc as plsc

H, W = 240, 304
K = 10
R = 3
TAU = 1000000.0
DELTA_T = 100000.0
GH = (H + K - 1) // K
GW = (W + K - 1) // K
NC = GH * GW
S = 2 * R + 1
B = 8
TPAD = 2048

NSTREAMS = 8
CPS = NC // NSTREAMS
BIN = 64
CNT_LANE = 49
GPAD = 40
GWORDS = CPS * 2 * K * K + 2 * GPAD + 8
HWORDS = CPS * 2 * BIN
HALL = HWORDS + BIN
RECW = 8
WCAP = TPAD + 4

_lut = np.zeros((K * K, BIN), np.float32)
for _ly in range(K):
    for _lx in range(K):
        for _s in range(S * S):
            _r, _c = _ly + _s // S - R, _lx + _s % S - R
            if 0 <= _r < K and 0 <= _c < K:
                _lut[_ly * K + _lx, _s] = 1.0
_LUT = _lut.reshape(-1)


def _body(ev_hbm, len_hbm, lut_hbm, out_hbm, ev_v, len_v, lut_v,
          g_a, g_b, h_a, h_b, wrec):
    ci = lax.axis_index("c")
    si = lax.axis_index("s")
    wid = si * 2 + ci
    b = wid // 4
    grp = wid % 4
    lo_a = (grp * 2) * CPS
    lo_b = lo_a + CPS

    pltpu.sync_copy(ev_hbm.at[b], ev_v)
    pltpu.sync_copy(len_hbm, len_v)
    pltpu.sync_copy(lut_hbm, lut_v)

    iota16 = lax.iota(jnp.int32, 16)
    zeros16 = (iota16 * 0).astype(jnp.float32)
    ones16 = zeros16 + 1.0
    lane0 = iota16 == 0
    lenvec = len_v[pl.ds(0, 16)]
    length = jnp.sum(jnp.where(iota16 == b, lenvec, 0))

    def zero8(ref, i, base):
        for u in range(8):
            ref[pl.ds(base + i * 128 + u * 16, 16)] = zeros16

    def zero_g(i, c):
        zero8(g_a, i, 0)
        zero8(g_b, i, 0)
        return c

    lax.fori_loop(0, GWORDS // 128, zero_g, 0)

    def zero_h(i, c):
        zero8(h_a, i, 0)
        zero8(h_b, i, 0)
        return c

    lax.fori_loop(0, HALL // 128, zero_h, 0)

    off_t = []
    for k in range(4):
        s = iota16 + 16 * k
        in49 = s < S * S
        off_t.append(jnp.where(in49, (lax.div(s, S) - R) * K + (lax.rem(s, S) - R), 0))
    cnt_t = ((iota16 + 48) == CNT_LANE).astype(jnp.float32)

    def scan(k, offs):
        off_sa, off_sb = offs
        xi = ev_v[0, pl.ds(k * 16, 16)].astype(jnp.int32)
        yi = ev_v[1, pl.ds(k * 16, 16)].astype(jnp.int32)
        tv = ev_v[2, pl.ds(k * 16, 16)]
        pi = ev_v[3, pl.ds(k * 16, 16)].astype(jnp.int32)
        ch = lax.div(yi, K)
        cw = lax.div(xi, K)
        cid = ch * GW + cw
        lyv = yi - ch * K
        lxv = xi - cw * K
        idxv = k * 16 + iota16
        valid = idxv < length
        e_v = jnp.exp(tv * (1.0 / TAU))
        inv_v = jnp.exp(tv * (-1.0 / TAU))
        cp = cid * 2 + pi
        gq = cp * (K * K) + lyv * K + lxv + GPAD
        hq = cp * BIN
        mbase = (lyv * K + lxv) * BIN

        m_a = valid & (cid >= lo_a) & (cid < lo_a + CPS)
        m_b = valid & (cid >= lo_b) & (cid < lo_b + CPS)
        cs_a = jnp.cumsum(m_a.astype(jnp.int32))
        cs_b = jnp.cumsum(m_b.astype(jnp.int32))
        pos_a = (off_sa + cs_a - 1) * RECW
        pos_b = (WCAP * RECW) + (off_sb + cs_b - 1) * RECW
        m = m_a | m_b
        pos = jnp.where(m_a, pos_a, pos_b)
        lo = jnp.where(m_a, lo_a, lo_b)
        plsc.store_scatter(wrec, [pos], tv, mask=m)
        plsc.store_scatter(wrec, [pos + 1], e_v, mask=m)
        plsc.store_scatter(wrec, [pos + 2], inv_v, mask=m)
        plsc.store_scatter(wrec, [pos + 3],
                           plsc.bitcast(gq - lo * (2 * K * K), jnp.float32),
                           mask=m)
        plsc.store_scatter(wrec, [pos + 4],
                           plsc.bitcast(hq - lo * (2 * BIN), jnp.float32),
                           mask=m)
        plsc.store_scatter(wrec, [pos + 5],
                           plsc.bitcast(mbase, jnp.float32), mask=m)
        return (off_sa + jnp.full((16,), cs_a[15], jnp.int32),
                off_sb + jnp.full((16,), cs_b[15], jnp.int32))

    zi = iota16 * 0
    nwv_a, nwv_b = lax.fori_loop(0, TPAD // 16, scan, (zi, zi))
    nw_a = nwv_a[0]
    nw_b = nwv_b[0]

    r8 = lax.rem(iota16, 8)
    fpart = jnp.where(r8 == 0, -1e30, jnp.where(r8 == 1, 1.0, 0.0))
    ipart = jnp.where(r8 == 3, GPAD, jnp.where(r8 == 4, HWORDS, 0))
    dummy = jnp.where(r8 < 3, fpart, plsc.bitcast(ipart, jnp.float32))
    wrec[pl.ds(nw_a * RECW, 16)] = dummy
    wrec[pl.ds((nw_a + 2) * RECW, 16)] = dummy
    wrec[pl.ds((WCAP + nw_b) * RECW, 16)] = dummy
    wrec[pl.ds((WCAP + nw_b + 2) * RECW, 16)] = dummy

    ctr_t = ((iota16 + 16) == (R * S + R)).astype(jnp.float32)

    def step(rbase, g_v, h_v, i, nw, carry):
        L, texp = carry
        ii = rbase + jnp.minimum(4 * i, nw)
        ii2 = rbase + jnp.minimum(4 * i + 2, nw)
        va = wrec[pl.ds(ii * RECW, 16)]
        vb = wrec[pl.ds(ii2 * RECW, 16)]
        via = plsc.bitcast(va, jnp.int32)
        vib = plsc.bitcast(vb, jnp.int32)
        for v16, vi16, half in ((va, via, 0), (va, via, 8),
                                (vb, vib, 0), (vb, vib, 8)):
            cutoff = v16[half] - DELTA_T

            def cond(c):
                return c[1] < cutoff

            def expire(c):
                Lc, _ = c
                rv = wrec[pl.ds((rbase + Lc) * RECW, 16)]
                rvi = plsc.bitcast(rv, jnp.int32)
                plsc.addupdate_scatter(
                    g_v, [jnp.full((16,), rvi[3], jnp.int32)],
                    zeros16 - jnp.full((16,), rv[1], jnp.float32), mask=lane0)
                nxt = wrec[pl.ds((rbase + Lc + 1) * RECW, 16)]
                return (Lc + 1, nxt[0])

            L, texp = lax.while_loop(cond, expire, (L, texp))

            giv = jnp.full((16,), vi16[half + 3], jnp.int32)
            invv = jnp.full((16,), v16[half + 2], jnp.float32)
            hbv = jnp.full((16,), vi16[half + 4], jnp.int32) + iota16
            mbv = jnp.full((16,), vi16[half + 5], jnp.int32) + iota16
            for k in range(4):
                gval = plsc.load_gather(g_v, [giv + off_t[k]])
                mk = plsc.load_gather(lut_v, [mbv + 16 * k])
                vals = gval * (mk * invv)
                if k == 1:
                    vals = vals + ctr_t
                if k == 3:
                    vals = vals + cnt_t
                plsc.addupdate_scatter(h_v, [hbv + 16 * k], vals)
            plsc.addupdate_scatter(
                g_v, [giv], jnp.full((16,), v16[half + 1], jnp.float32),
                mask=lane0)
        return (L, texp)

    texp_a = wrec[pl.ds(0, 16)][0]
    texp_b = wrec[pl.ds(WCAP * RECW, 16)][0]

    def proc(i, carry):
        ca, cb = carry
        ca = step(0, g_a, h_a, i, nw_a, ca)
        cb = step(WCAP, g_b, h_b, i, nw_b, cb)
        return (ca, cb)

    lax.fori_loop(0, lax.div(jnp.maximum(nw_a, nw_b) + 3, 4), proc,
                  ((jnp.int32(0), texp_a), (jnp.int32(0), texp_b)))

    def norm1(h_v, c):
        cnt = (h_v[pl.ds(c * (2 * BIN) + 48, 16)][CNT_LANE - 48]
               + h_v[pl.ds(c * (2 * BIN) + BIN + 48, 16)][CNT_LANE - 48])
        scale = ones16 / jnp.full((16,), cnt + 1e-6, jnp.float32)
        for k in range(2 * BIN // 16):
            sl = pl.ds(c * (2 * BIN) + k * 16, 16)
            h_v[sl] = h_v[sl] * scale

    def norm(c, carry):
        norm1(h_a, c)
        norm1(h_b, c)
        return carry

    lax.fori_loop(0, CPS, norm, 0)

    base = (b * NC + lo_a) * (2 * BIN)
    pltpu.sync_copy(h_a.at[pl.ds(0, HWORDS)], out_hbm.at[pl.ds(base, HWORDS)])
    pltpu.sync_copy(h_b.at[pl.ds(0, HWORDS)],
                    out_hbm.at[pl.ds(base + HWORDS, HWORDS)])


@jax.jit
def _hats_sc(comp, len16, lut):
    mesh = plsc.VectorSubcoreMesh(core_axis_name="c", subcore_axis_name="s",
                                  num_cores=2, num_subcores=16)
    f = pl.kernel(
        _body,
        out_type=jax.ShapeDtypeStruct((B * NC * 2 * BIN,), jnp.float32),
        mesh=mesh,
        compiler_params=pltpu.CompilerParams(needs_layout_passes=False),
        scratch_types=[
            pltpu.VMEM((4, TPAD), jnp.float32),
            pltpu.VMEM((16,), jnp.int32),
            pltpu.VMEM((K * K * BIN,), jnp.float32),
            pltpu.VMEM((GWORDS,), jnp.float32),
            pltpu.VMEM((GWORDS,), jnp.float32),
            pltpu.VMEM((HALL,), jnp.float32),
            pltpu.VMEM((HALL,), jnp.float32),
            pltpu.VMEM((2 * WCAP * RECW,), jnp.float32),
        ],
    )
    return f(comp, len16, lut)


def kernel(events, lengths):
    comp = jnp.transpose(events, (0, 2, 1))
    len16 = jnp.zeros((16,), jnp.int32).at[:B].set(lengths.astype(jnp.int32))
    flat = _hats_sc(comp, len16, jnp.asarray(_LUT))
    out = flat.reshape(B, NC, 2, BIN)[..., :S * S]
    return out.reshape(B, NC, 2, S, S)
